# Initial kernel scaffold; baseline (speedup 1.0000x reference)
import jax
import jax.numpy as jnp
from jax.experimental import pallas as pl


def kernel(x, edge_index, batch, conv0_w, conv0_b, conv1_w, conv1_b, conv2_w, conv2_b, conv3_w, conv3_b, fc_max_w, fc_max_b, fc1_w, fc1_b, fc2_w, fc2_b):
    raise NotImplementedError("write your pallas kernel here")


# speedup vs baseline: 28.8732x; 28.8732x over previous
import functools

import jax
import jax.numpy as jnp
from jax import lax
from jax.experimental import pallas as pl
from jax.experimental.pallas import tpu as pltpu
from jax.experimental.pallas import tpu_sc as plsc

N = 10000
E = 320000
DF = 128
DE = 64
NG = 64
DT = 10
DH = 4 * DE

GROUP = 200
EDGES_PER_TILE = E // 32
GROUPS_PER_TILE = EDGES_PER_TILE // GROUP
NROWS_TILE = N // 16

_mesh = plsc.VectorSubcoreMesh(core_axis_name="c", subcore_axis_name="s",
                               num_cores=2, num_subcores=16)
_sc_params = pltpu.CompilerParams(use_tc_tiling_on_sc=False)


def _fill3d(ref, rows, cols, value):
    vals = jnp.full((16,), value, jnp.float32)

    def body(i, carry):
        for k in range(cols // 16):
            ref[0, i, pl.ds(k * 16, 16)] = vals
        return carry

    lax.fori_loop(0, rows, body, 0, unroll=False)


def _zero_slice(zrows_ref, dst_sh, base, total):
    off = 0
    while off < total:
        n = min(GROUP, total - off)
        pltpu.sync_copy(zrows_ref.at[0, pl.ds(0, n)],
                        dst_sh.at[0, pl.ds(base + off, n)])
        off += n


@functools.partial(
    pl.kernel,
    out_type=jax.ShapeDtypeStruct((2, N, 16), jnp.float32),
    mesh=_mesh,
    compiler_params=_sc_params,
    scratch_types=[
        pltpu.VMEM((1, GROUP, 16), jnp.float32),
        pltpu.VMEM((GROUPS_PER_TILE, 1, GROUP), jnp.int32),
        pltpu.VMEM_SHARED((1, N, 16), jnp.float32),
    ],
)
def _deg_call(dst_hbm, out_hbm, ones_b, didx, degw_sh):
    c = lax.axis_index("c")
    s = lax.axis_index("s")
    w = c * 16 + s
    _fill3d(ones_b, GROUP, 16, 0.0)
    _zero_slice(ones_b, degw_sh, s * NROWS_TILE, NROWS_TILE)
    _fill3d(ones_b, GROUP, 16, 1.0)
    pltpu.sync_copy(dst_hbm.at[pl.ds(w * GROUPS_PER_TILE, GROUPS_PER_TILE)], didx)
    plsc.subcore_barrier()

    def body(g, carry):
        pltpu.sync_copy(ones_b, degw_sh.at[didx.at[g]], add=True)
        return carry

    lax.fori_loop(0, GROUPS_PER_TILE, body, 0, unroll=False)
    plsc.subcore_barrier()
    sl = pl.ds(s * NROWS_TILE, NROWS_TILE)
    pltpu.sync_copy(degw_sh.at[0, sl], out_hbm.at[c, sl])


@functools.partial(
    pl.kernel,
    out_type=jax.ShapeDtypeStruct((2, N, DE), jnp.float32),
    mesh=_mesh,
    compiler_params=_sc_params,
    scratch_types=[
        pltpu.VMEM((2, 1, GROUP, DE), jnp.float32),
        pltpu.VMEM((GROUPS_PER_TILE, 1, GROUP), jnp.int32),
        pltpu.VMEM((GROUPS_PER_TILE, 1, GROUP), jnp.int32),
        pltpu.VMEM_SHARED((1, N, DE), jnp.float32),
        pltpu.VMEM_SHARED((1, N, DE), jnp.float32),
        pltpu.SemaphoreType.DMA,
        pltpu.SemaphoreType.DMA,
        pltpu.SemaphoreType.DMA,
        pltpu.SemaphoreType.DMA,
    ],
)
def _agg_call(h_hbm, src_hbm, dst_hbm, out_hbm, rows, sidx, didx, agg_sh,
              hst_sh, gsem0, gsem1, ssem0, ssem1):
    c = lax.axis_index("c")
    s = lax.axis_index("s")
    w = c * 16 + s
    gsems = (gsem0, gsem1)
    ssems = (ssem0, ssem1)
    hsl = pl.ds(s * NROWS_TILE, NROWS_TILE)
    pltpu.sync_copy(h_hbm.at[0, hsl], hst_sh.at[0, hsl])
    _fill3d(rows.at[0], GROUP, DE, 0.0)
    _zero_slice(rows.at[0], agg_sh, s * NROWS_TILE, NROWS_TILE)
    pltpu.sync_copy(src_hbm.at[pl.ds(w * GROUPS_PER_TILE, GROUPS_PER_TILE)], sidx)
    pltpu.sync_copy(dst_hbm.at[pl.ds(w * GROUPS_PER_TILE, GROUPS_PER_TILE)], didx)
    plsc.subcore_barrier()

    def pair_body(i, carry):
        for b in range(2):
            g = 2 * i + b

            @pl.when(g >= 2)
            def _():
                pltpu.make_async_copy(
                    rows.at[b], agg_sh.at[didx.at[g - 2]], ssems[b]).wait()

            pltpu.async_copy(hst_sh.at[sidx.at[g]], rows.at[b], gsems[b]).wait()
            pltpu.async_copy(rows.at[b], agg_sh.at[didx.at[g]], ssems[b],
                             add=True)
        return carry

    lax.fori_loop(0, GROUPS_PER_TILE // 2, pair_body, 0, unroll=False)
    for b in range(2):
        pltpu.make_async_copy(
            rows.at[b],
            agg_sh.at[didx.at[GROUPS_PER_TILE - 2 + b]], ssems[b]).wait()
    plsc.subcore_barrier()
    sl = pl.ds(s * NROWS_TILE, NROWS_TILE)
    pltpu.sync_copy(agg_sh.at[0, sl], out_hbm.at[c, sl])


BLK = 1000


def _prep_body(deg_ref, x_ref, w_ref, dinv_ref, hhat_ref):
    deg = deg_ref[0, :, 0] + deg_ref[1, :, 0] + 1.0
    dinv = lax.rsqrt(deg)[:, None]
    dinv64 = jnp.broadcast_to(dinv, (BLK, DE))
    dinv_ref[...] = dinv64
    h0 = jnp.dot(x_ref[...], w_ref[...], preferred_element_type=jnp.float32)
    hhat_ref[...] = dinv64 * h0


_prep = pl.pallas_call(
    _prep_body,
    grid=(N // BLK,),
    in_specs=[
        pl.BlockSpec((2, BLK, 16), lambda i: (0, i, 0)),
        pl.BlockSpec((BLK, DF), lambda i: (i, 0)),
        pl.BlockSpec((DF, DE), lambda i: (0, 0)),
    ],
    out_specs=[
        pl.BlockSpec((BLK, DE), lambda i: (i, 0)),
        pl.BlockSpec((BLK, DE), lambda i: (i, 0)),
    ],
    out_shape=[
        jax.ShapeDtypeStruct((N, DE), jnp.float32),
        jax.ShapeDtypeStruct((N, DE), jnp.float32),
    ],
)


def _layer_body(aggp_ref, hhat_ref, dinv_ref, b_ref, fmw_ref, fmb_ref,
                wn_ref, hnew_ref, hhatn_ref):
    agg = aggp_ref[0] + aggp_ref[1] + hhat_ref[...]
    conv = dinv_ref[...] * agg + b_ref[...]
    hnew = jnp.maximum(
        jnp.dot(conv, fmw_ref[...], preferred_element_type=jnp.float32)
        + fmb_ref[...], 0.0)
    hnew_ref[...] = hnew
    if wn_ref is not None:
        hhatn_ref[...] = dinv_ref[...] * jnp.dot(
            hnew, wn_ref[...], preferred_element_type=jnp.float32)


def _make_layer(with_next):
    in_specs = [
        pl.BlockSpec((2, BLK, DE), lambda i: (0, i, 0)),
        pl.BlockSpec((BLK, DE), lambda i: (i, 0)),
        pl.BlockSpec((BLK, DE), lambda i: (i, 0)),
        pl.BlockSpec((1, DE), lambda i: (0, 0)),
        pl.BlockSpec((DE, DE), lambda i: (0, 0)),
        pl.BlockSpec((1, DE), lambda i: (0, 0)),
    ]
    out_specs = [pl.BlockSpec((BLK, DE), lambda i: (i, 0))]
    out_shape = [jax.ShapeDtypeStruct((N, DE), jnp.float32)]
    if with_next:
        in_specs.append(pl.BlockSpec((DE, DE), lambda i: (0, 0)))
        out_specs.append(pl.BlockSpec((BLK, DE), lambda i: (i, 0)))
        out_shape.append(jax.ShapeDtypeStruct((N, DE), jnp.float32))
        body = _layer_body
    else:
        def body(aggp, hhat, dinv, b, fmw, fmb, hnew):
            _layer_body(aggp, hhat, dinv, b, fmw, fmb, None, hnew, None)
    return pl.pallas_call(
        body, grid=(N // BLK,), in_specs=in_specs, out_specs=out_specs,
        out_shape=out_shape)


_layer_mid = _make_layer(True)
_layer_last = _make_layer(False)


def _pool_body(hcat_ref, batch_ref, out_ref):
    g0 = batch_ref[0, 0]
    g1 = batch_ref[BLK - 1, 0]
    ids = batch_ref[...]
    hc = hcat_ref[...]
    gid2d = lax.broadcasted_iota(jnp.int32, (NG, DH), 0)

    def body(g, acc):
        masked = jnp.where(ids == g, hc, 0.0)
        colmax = jnp.max(masked, axis=0)[None, :]
        return jnp.where(gid2d == g, jnp.maximum(acc, colmax), acc)

    acc = lax.fori_loop(g0, g1 + 1, body, jnp.zeros((NG, DH), jnp.float32))
    out_ref[0] = acc


_pool = pl.pallas_call(
    _pool_body,
    grid=(N // BLK,),
    in_specs=[
        pl.BlockSpec((BLK, DH), lambda i: (i, 0)),
        pl.BlockSpec((BLK, 1), lambda i: (i, 0)),
    ],
    out_specs=pl.BlockSpec((1, NG, DH), lambda i: (i, 0, 0)),
    out_shape=jax.ShapeDtypeStruct((N // BLK, NG, DH), jnp.float32),
)


def _final_body(pooledp_ref, w1_ref, b1_ref, w2_ref, b2_ref, out_ref):
    pooled = jnp.max(pooledp_ref[...], axis=0)
    z = jnp.maximum(
        jnp.dot(pooled, w1_ref[...], preferred_element_type=jnp.float32)
        + b1_ref[...], 0.0)
    out_ref[...] = jnp.dot(z, w2_ref[...],
                           preferred_element_type=jnp.float32) + b2_ref[...]


_final = pl.pallas_call(
    _final_body,
    out_shape=jax.ShapeDtypeStruct((NG, 128), jnp.float32),
)


def kernel(x, edge_index, batch, conv0_w, conv0_b, conv1_w, conv1_b,
           conv2_w, conv2_b, conv3_w, conv3_b, fc_max_w, fc_max_b,
           fc1_w, fc1_b, fc2_w, fc2_b):
    src_p = edge_index[0].reshape(E // GROUP, 1, GROUP)
    dst_p = edge_index[1].reshape(E // GROUP, 1, GROUP)

    deg_p = _deg_call(dst_p)
    dinv64, hhat = _prep(deg_p, x, conv0_w)

    conv_b = [conv0_b, conv1_b, conv2_b, conv3_b]
    next_w = [conv1_w, conv2_w, conv3_w]
    fmb = fc_max_b.reshape(1, DE)
    hs = []
    for l in range(4):
        aggp = _agg_call(hhat.reshape(1, N, DE), src_p, dst_p)
        bl = conv_b[l].reshape(1, DE)
        if l < 3:
            hnew, hhat = _layer_mid(aggp, hhat, dinv64, bl, fc_max_w, fmb,
                                    next_w[l])
        else:
            (hnew,) = _layer_last(aggp, hhat, dinv64, bl, fc_max_w, fmb)
        hs.append(hnew)

    hcat = jnp.concatenate(hs, axis=1)
    pooledp = _pool(hcat, batch.reshape(N, 1))

    w2 = jnp.pad(fc2_w, ((0, 0), (0, 128 - DT)))
    b2 = jnp.pad(fc2_b, (0, 128 - DT)).reshape(1, 128)
    out = _final(pooledp, fc1_w, fc1_b.reshape(1, DE), w2, b2)
    return out[:, :DT]


# speedup vs baseline: 29.4713x; 1.0207x over previous
import functools

import jax
import jax.numpy as jnp
from jax import lax
from jax.experimental import pallas as pl
from jax.experimental.pallas import tpu as pltpu
from jax.experimental.pallas import tpu_sc as plsc

N = 10000
E = 320000
DF = 128
DE = 64
NG = 64
DT = 10
DH = 4 * DE

GROUP = 200
EDGES_PER_TILE = E // 32
GROUPS_PER_TILE = EDGES_PER_TILE // GROUP
NROWS_TILE = N // 16

_mesh = plsc.VectorSubcoreMesh(core_axis_name="c", subcore_axis_name="s",
                               num_cores=2, num_subcores=16)
_sc_params = pltpu.CompilerParams(use_tc_tiling_on_sc=False)


def _fill3d(ref, rows, cols, value):
    vals = jnp.full((16,), value, jnp.float32)

    def body(i, carry):
        for k in range(cols // 16):
            ref[0, i, pl.ds(k * 16, 16)] = vals
        return carry

    lax.fori_loop(0, rows, body, 0, unroll=False)


def _zero_slice(zrows_ref, dst_sh, base, total):
    off = 0
    while off < total:
        n = min(GROUP, total - off)
        pltpu.sync_copy(zrows_ref.at[0, pl.ds(0, n)],
                        dst_sh.at[0, pl.ds(base + off, n)])
        off += n


@functools.partial(
    pl.kernel,
    out_type=jax.ShapeDtypeStruct((2, N, 16), jnp.float32),
    mesh=_mesh,
    compiler_params=_sc_params,
    scratch_types=[
        pltpu.VMEM((1, GROUP, 16), jnp.float32),
        pltpu.VMEM((GROUPS_PER_TILE, 1, GROUP), jnp.int32),
        pltpu.VMEM_SHARED((1, N, 16), jnp.float32),
    ],
)
def _deg_call(dst_hbm, out_hbm, ones_b, didx, degw_sh):
    c = lax.axis_index("c")
    s = lax.axis_index("s")
    w = c * 16 + s
    _fill3d(ones_b, GROUP, 16, 0.0)
    _zero_slice(ones_b, degw_sh, s * NROWS_TILE, NROWS_TILE)
    _fill3d(ones_b, GROUP, 16, 1.0)
    pltpu.sync_copy(dst_hbm.at[pl.ds(w * GROUPS_PER_TILE, GROUPS_PER_TILE)], didx)
    plsc.subcore_barrier()

    def body(g, carry):
        pltpu.sync_copy(ones_b, degw_sh.at[didx.at[g]], add=True)
        return carry

    lax.fori_loop(0, GROUPS_PER_TILE, body, 0, unroll=False)
    plsc.subcore_barrier()
    sl = pl.ds(s * NROWS_TILE, NROWS_TILE)
    pltpu.sync_copy(degw_sh.at[0, sl], out_hbm.at[c, sl])


@functools.partial(
    pl.kernel,
    out_type=jax.ShapeDtypeStruct((2, N, DE), jnp.float32),
    mesh=_mesh,
    compiler_params=_sc_params,
    scratch_types=[
        pltpu.VMEM((2, 1, GROUP, DE), jnp.float32),
        pltpu.VMEM((GROUPS_PER_TILE, 1, GROUP), jnp.int32),
        pltpu.VMEM((GROUPS_PER_TILE, 1, GROUP), jnp.int32),
        pltpu.VMEM_SHARED((1, N, DE), jnp.float32),
        pltpu.VMEM_SHARED((1, N, DE), jnp.float32),
        pltpu.SemaphoreType.DMA,
        pltpu.SemaphoreType.DMA,
        pltpu.SemaphoreType.DMA,
        pltpu.SemaphoreType.DMA,
    ],
)
def _agg_call(h_hbm, src_hbm, dst_hbm, out_hbm, rows, sidx, didx, agg_sh,
              hst_sh, gsem0, gsem1, ssem0, ssem1):
    c = lax.axis_index("c")
    s = lax.axis_index("s")
    w = c * 16 + s
    gsems = (gsem0, gsem1)
    ssems = (ssem0, ssem1)
    hsl = pl.ds(s * NROWS_TILE, NROWS_TILE)
    pltpu.sync_copy(h_hbm.at[0, hsl], hst_sh.at[0, hsl])
    _fill3d(rows.at[0], GROUP, DE, 0.0)
    _zero_slice(rows.at[0], agg_sh, s * NROWS_TILE, NROWS_TILE)
    pltpu.sync_copy(src_hbm.at[pl.ds(w * GROUPS_PER_TILE, GROUPS_PER_TILE)], sidx)
    pltpu.sync_copy(dst_hbm.at[pl.ds(w * GROUPS_PER_TILE, GROUPS_PER_TILE)], didx)
    plsc.subcore_barrier()

    def pair_body(i, carry):
        for b in range(2):
            g = 2 * i + b

            @pl.when(g >= 2)
            def _():
                pltpu.make_async_copy(
                    rows.at[b], agg_sh.at[didx.at[g - 2]], ssems[b]).wait()

            pltpu.async_copy(hst_sh.at[sidx.at[g]], rows.at[b], gsems[b]).wait()
            pltpu.async_copy(rows.at[b], agg_sh.at[didx.at[g]], ssems[b],
                             add=True)
        return carry

    lax.fori_loop(0, GROUPS_PER_TILE // 2, pair_body, 0, unroll=False)
    for b in range(2):
        pltpu.make_async_copy(
            rows.at[b],
            agg_sh.at[didx.at[GROUPS_PER_TILE - 2 + b]], ssems[b]).wait()
    plsc.subcore_barrier()
    sl = pl.ds(s * NROWS_TILE, NROWS_TILE)
    pltpu.sync_copy(agg_sh.at[0, sl], out_hbm.at[c, sl])


BLK = 1000


def _prep_body(deg_ref, x_ref, w_ref, dinv_ref, hhat_ref):
    deg = deg_ref[0, :, 0] + deg_ref[1, :, 0] + 1.0
    dinv = lax.rsqrt(deg)[:, None]
    dinv64 = jnp.broadcast_to(dinv, (BLK, DE))
    dinv_ref[...] = dinv64
    h0 = jnp.dot(x_ref[...], w_ref[...], preferred_element_type=jnp.float32)
    hhat_ref[...] = dinv64 * h0


_prep = pl.pallas_call(
    _prep_body,
    grid=(N // BLK,),
    in_specs=[
        pl.BlockSpec((2, BLK, 16), lambda i: (0, i, 0)),
        pl.BlockSpec((BLK, DF), lambda i: (i, 0)),
        pl.BlockSpec((DF, DE), lambda i: (0, 0)),
    ],
    out_specs=[
        pl.BlockSpec((BLK, DE), lambda i: (i, 0)),
        pl.BlockSpec((BLK, DE), lambda i: (i, 0)),
    ],
    out_shape=[
        jax.ShapeDtypeStruct((N, DE), jnp.float32),
        jax.ShapeDtypeStruct((N, DE), jnp.float32),
    ],
)


def _layer_body(aggp_ref, hhat_ref, dinv_ref, b_ref, fmw_ref, fmb_ref,
                wn_ref, hnew_ref, hhatn_ref):
    agg = aggp_ref[0] + aggp_ref[1] + hhat_ref[...]
    conv = dinv_ref[...] * agg + b_ref[...]
    hnew = jnp.maximum(
        jnp.dot(conv, fmw_ref[...], preferred_element_type=jnp.float32)
        + fmb_ref[...], 0.0)
    hnew_ref[...] = hnew
    if wn_ref is not None:
        hhatn_ref[...] = dinv_ref[...] * jnp.dot(
            hnew, wn_ref[...], preferred_element_type=jnp.float32)


def _make_layer(with_next):
    in_specs = [
        pl.BlockSpec((2, BLK, DE), lambda i: (0, i, 0)),
        pl.BlockSpec((BLK, DE), lambda i: (i, 0)),
        pl.BlockSpec((BLK, DE), lambda i: (i, 0)),
        pl.BlockSpec((1, DE), lambda i: (0, 0)),
        pl.BlockSpec((DE, DE), lambda i: (0, 0)),
        pl.BlockSpec((1, DE), lambda i: (0, 0)),
    ]
    out_specs = [pl.BlockSpec((BLK, DE), lambda i: (i, 0))]
    out_shape = [jax.ShapeDtypeStruct((N, DE), jnp.float32)]
    if with_next:
        in_specs.append(pl.BlockSpec((DE, DE), lambda i: (0, 0)))
        out_specs.append(pl.BlockSpec((BLK, DE), lambda i: (i, 0)))
        out_shape.append(jax.ShapeDtypeStruct((N, DE), jnp.float32))
        body = _layer_body
    else:
        def body(aggp, hhat, dinv, b, fmw, fmb, hnew):
            _layer_body(aggp, hhat, dinv, b, fmw, fmb, None, hnew, None)
    return pl.pallas_call(
        body, grid=(N // BLK,), in_specs=in_specs, out_specs=out_specs,
        out_shape=out_shape)


_layer_mid = _make_layer(True)
_layer_last = _make_layer(False)


def _pool_body(h0_ref, h1_ref, h2_ref, h3_ref, batch_ref, out_ref):
    g0 = batch_ref[0, 0]
    g1 = batch_ref[BLK - 1, 0]
    ids = batch_ref[...]
    hc = jnp.concatenate(
        [h0_ref[...], h1_ref[...], h2_ref[...], h3_ref[...]], axis=1)
    gid2d = lax.broadcasted_iota(jnp.int32, (NG, DH), 0)

    def body(g, acc):
        masked = jnp.where(ids == g, hc, 0.0)
        colmax = jnp.max(masked, axis=0)[None, :]
        return jnp.where(gid2d == g, jnp.maximum(acc, colmax), acc)

    acc = lax.fori_loop(g0, g1 + 1, body, jnp.zeros((NG, DH), jnp.float32))
    out_ref[0] = acc


_pool = pl.pallas_call(
    _pool_body,
    grid=(N // BLK,),
    in_specs=[
        pl.BlockSpec((BLK, DE), lambda i: (i, 0)),
        pl.BlockSpec((BLK, DE), lambda i: (i, 0)),
        pl.BlockSpec((BLK, DE), lambda i: (i, 0)),
        pl.BlockSpec((BLK, DE), lambda i: (i, 0)),
        pl.BlockSpec((BLK, 1), lambda i: (i, 0)),
    ],
    out_specs=pl.BlockSpec((1, NG, DH), lambda i: (i, 0, 0)),
    out_shape=jax.ShapeDtypeStruct((N // BLK, NG, DH), jnp.float32),
)


def _final_body(pooledp_ref, w1_ref, b1_ref, w2_ref, b2_ref, out_ref):
    pooled = jnp.max(pooledp_ref[...], axis=0)
    z = jnp.maximum(
        jnp.dot(pooled, w1_ref[...], preferred_element_type=jnp.float32)
        + b1_ref[...], 0.0)
    out_ref[...] = jnp.dot(z, w2_ref[...],
                           preferred_element_type=jnp.float32) + b2_ref[...]


_final = pl.pallas_call(
    _final_body,
    out_shape=jax.ShapeDtypeStruct((NG, 128), jnp.float32),
)


def kernel(x, edge_index, batch, conv0_w, conv0_b, conv1_w, conv1_b,
           conv2_w, conv2_b, conv3_w, conv3_b, fc_max_w, fc_max_b,
           fc1_w, fc1_b, fc2_w, fc2_b):
    src_p = edge_index[0].reshape(E // GROUP, 1, GROUP)
    dst_p = edge_index[1].reshape(E // GROUP, 1, GROUP)

    deg_p = _deg_call(dst_p)
    dinv64, hhat = _prep(deg_p, x, conv0_w)

    conv_b = [conv0_b, conv1_b, conv2_b, conv3_b]
    next_w = [conv1_w, conv2_w, conv3_w]
    fmb = fc_max_b.reshape(1, DE)
    hs = []
    for l in range(4):
        aggp = _agg_call(hhat.reshape(1, N, DE), src_p, dst_p)
        bl = conv_b[l].reshape(1, DE)
        if l < 3:
            hnew, hhat = _layer_mid(aggp, hhat, dinv64, bl, fc_max_w, fmb,
                                    next_w[l])
        else:
            (hnew,) = _layer_last(aggp, hhat, dinv64, bl, fc_max_w, fmb)
        hs.append(hnew)

    pooledp = _pool(hs[0], hs[1], hs[2], hs[3],
                    batch.reshape(N, 1))

    w2 = jnp.pad(fc2_w, ((0, 0), (0, 128 - DT)))
    b2 = jnp.pad(fc2_b, (0, 128 - DT)).reshape(1, 128)
    out = _final(pooledp, fc1_w, fc1_b.reshape(1, DE), w2, b2)
    return out[:, :DT]
